# baseline (speedup 1.0000x reference)
import jax
import jax.numpy as jnp
from jax.experimental import pallas as pl
from jax.experimental.pallas import tpu as pltpu

HIDDEN = 4096
PLEN = 10
BATCH = 4


def _body(idx_ref, w_ref, prompt_ref, mode_ref):
    x = w_ref[...]
    mode_ref[...] = x
    prompt_ref[...] = jnp.broadcast_to(x, (BATCH, PLEN, HIDDEN))


def kernel(mode_embeddings_weight, mode_idx, batch_size):
    del batch_size
    w3 = mode_embeddings_weight.reshape(-1, PLEN, HIDDEN)
    idx = jnp.atleast_1d(mode_idx).astype(jnp.int32)
    grid_spec = pltpu.PrefetchScalarGridSpec(
        num_scalar_prefetch=1,
        grid=(1,),
        in_specs=[
            pl.BlockSpec((1, PLEN, HIDDEN), lambda i, idx_ref: (idx_ref[0], 0, 0)),
        ],
        out_specs=[
            pl.BlockSpec((BATCH, PLEN, HIDDEN), lambda i, idx_ref: (0, 0, 0)),
            pl.BlockSpec((1, PLEN, HIDDEN), lambda i, idx_ref: (0, 0, 0)),
        ],
    )
    prompt, mode3 = pl.pallas_call(
        _body,
        grid_spec=grid_spec,
        out_shape=[
            jax.ShapeDtypeStruct((BATCH, PLEN, HIDDEN), jnp.float32),
            jax.ShapeDtypeStruct((1, PLEN, HIDDEN), jnp.float32),
        ],
    )(idx, w3)
    return prompt, mode3.reshape(1, PLEN * HIDDEN)
